# baseline (speedup 1.0000x reference)
import functools
import math

import jax
import jax.numpy as jnp
from jax import lax
from jax.experimental import pallas as pl
from jax.experimental.pallas import tpu as pltpu
from jax.experimental.pallas import tpu_sc as plsc

N = 8388608
EPS = 1e-6
B = 1792
BR, BL = 14, 128
B16 = B * 16
NT = 32
ROWS = 8192
NROW = ROWS // NT
RCH = 16
NCH = NROW // RCH
BLK = 512
GRID1 = ROWS // BLK
CF = float(1.63 * math.sqrt(8388608.0))


def _stats_body(p_ref, t_ref, scal_ref, par_ref,
                a_t, a_sq, a_mx, a_mn):
    i = pl.program_id(0)
    p = p_ref[...]
    t = t_ref[...]
    d = p - t
    bs_t = jnp.sum(t, axis=0, keepdims=True)
    bs_sq = jnp.sum(d * d, axis=0, keepdims=True)
    b_mx = jnp.max(jnp.maximum(p, t), axis=0, keepdims=True)
    b_mn = jnp.min(jnp.minimum(p, t), axis=0, keepdims=True)

    @pl.when(i == 0)
    def _():
        a_t[...] = bs_t
        a_sq[...] = bs_sq
        a_mx[...] = b_mx
        a_mn[...] = b_mn

    @pl.when(i > 0)
    def _():
        a_t[...] = a_t[...] + bs_t
        a_sq[...] = a_sq[...] + bs_sq
        a_mx[...] = jnp.maximum(a_mx[...], b_mx)
        a_mn[...] = jnp.minimum(a_mn[...], b_mn)

    @pl.when(i == pl.num_programs(0) - 1)
    def _():
        sum_t = jnp.sum(a_t[...])
        mse = jnp.sum(a_sq[...]) / N
        vmax = jnp.max(a_mx[...])
        vmin = jnp.min(a_mn[...])
        nrmse = jnp.sqrt(mse) / (sum_t / N + EPS)
        vc = vmax - vmin + EPS
        c = jnp.float32(CF) * vc / N
        scal_ref[0] = nrmse
        scal_ref[1] = 100.0 / (jnp.float32(CF) * vc)
        scal_ref[2] = c
        scal_ref[3] = vmin
        ii = lax.broadcasted_iota(jnp.int32, (1, 64), 1)
        par_ref[...] = jnp.where(ii < 16, vmin,
                                 jnp.where(ii < 32, 1.0 / c, 0.0))


def _stats(p2d, t2d):
    return pl.pallas_call(
        _stats_body,
        grid=(GRID1,),
        in_specs=[pl.BlockSpec((BLK, 1024), lambda i: (i, 0)),
                  pl.BlockSpec((BLK, 1024), lambda i: (i, 0))],
        out_specs=[pl.BlockSpec((4,), lambda i: (0,), memory_space=pltpu.SMEM),
                   pl.BlockSpec((1, 64), lambda i: (0, 0))],
        out_shape=[jax.ShapeDtypeStruct((4,), jnp.float32),
                   jax.ShapeDtypeStruct((1, 64), jnp.float32)],
        scratch_shapes=[pltpu.VMEM((1, 1024), jnp.float32)] * 4,
    )(p2d, t2d)


def _sc_body(p_hbm, t_hbm, par_hbm, out_hbm,
             hcnt, hsum, buf0, buf1, buf2, buf3, fold_v, par_v,
             sem0, sem1, sem2, sem3):
    wid = lax.axis_index("s") * 2 + lax.axis_index("c")
    row0 = wid * NROW
    pltpu.sync_copy(par_hbm, par_v)
    vmin = par_v[pl.ds(0, 16)]
    vinvh = par_v[pl.ds(16, 16)]
    lane = lax.iota(jnp.int32, 16)
    ones = jnp.full((16,), 1.0, jnp.float32)
    zeros = jnp.zeros((16,), jnp.float32)

    def zero_hists():
        def zbody(j, c):
            for u in range(4):
                hcnt[pl.ds(j * 64 + u * 16, 16)] = zeros
                hsum[pl.ds(j * 64 + u * 16, 16)] = zeros
            return c

        lax.fori_loop(0, B16 // 64, zbody, 0)

    def compute(buf):
        def vbody(k, c):
            vs = [buf[r, pl.ds(k * 16, 16)] for r in range(RCH)]
            ads = []
            for v in vs:
                idx = lax.convert_element_type((v - vmin) * vinvh, jnp.int32)
                ads.append(idx * 16 + lane)
            for v, a in zip(vs, ads):
                plsc.addupdate_scatter(hcnt, [a], ones)
                plsc.addupdate_scatter(hsum, [a], v)
            return c

        lax.fori_loop(0, 1024 // 16, vbody, 0)

    def src(x_hbm, ci):
        return x_hbm.at[pl.ds(row0 + ci * RCH, RCH), :]

    def process(x_hbm):
        ring = ((buf0, sem0), (buf1, sem1), (buf2, sem2), (buf3, sem3))
        nb = len(ring)
        for s, (buf, sem) in enumerate(ring):
            pltpu.make_async_copy(src(x_hbm, s), buf, sem).start()

        def cbody(cj, c):
            c0 = cj * nb
            for s, (buf, sem) in enumerate(ring):
                pltpu.make_async_copy(src(x_hbm, c0 + s), buf, sem).wait()
                compute(buf)

                @pl.when(c0 + s + nb < NCH)
                def _(buf=buf, sem=sem, s=s):
                    pltpu.make_async_copy(
                        src(x_hbm, c0 + s + nb), buf, sem).start()

            return c

        lax.fori_loop(0, NCH // nb, cbody, 0)

    def fold_out(orow):
        def fbody(j0, c):
            bkt = (j0 * 16 + lane) * 16
            pc = [plsc.load_gather(hcnt, [bkt + l]) for l in range(16)]
            ps = [plsc.load_gather(hsum, [bkt + l]) for l in range(16)]

            def tree(xs):
                while len(xs) > 1:
                    xs = [a + b for a, b in zip(xs[::2], xs[1::2])]
                return xs[0]

            fold_v[0, pl.ds(j0 * 16, 16)] = tree(pc)
            fold_v[1, pl.ds(j0 * 16, 16)] = tree(ps)
            return c

        lax.fori_loop(0, B // 16, fbody, 0)
        pltpu.sync_copy(fold_v, out_hbm.at[wid, pl.ds(orow, 2)])

    zero_hists()
    process(p_hbm)
    fold_out(0)
    zero_hists()
    process(t_hbm)
    fold_out(2)


def _sc_hist(p2d, t2d, params):
    mesh = plsc.VectorSubcoreMesh(core_axis_name="c", subcore_axis_name="s")
    f = functools.partial(
        pl.kernel,
        mesh=mesh,
        out_type=jax.ShapeDtypeStruct((NT, 4, B), jnp.float32),
        compiler_params=pltpu.CompilerParams(needs_layout_passes=False),
        scratch_types=[pltpu.VMEM((B16,), jnp.float32),
                       pltpu.VMEM((B16,), jnp.float32),
                       pltpu.VMEM((RCH, 1024), jnp.float32),
                       pltpu.VMEM((RCH, 1024), jnp.float32),
                       pltpu.VMEM((RCH, 1024), jnp.float32),
                       pltpu.VMEM((RCH, 1024), jnp.float32),
                       pltpu.VMEM((2, B), jnp.float32),
                       pltpu.VMEM((64,), jnp.float32),
                       pltpu.SemaphoreType.DMA,
                       pltpu.SemaphoreType.DMA,
                       pltpu.SemaphoreType.DMA,
                       pltpu.SemaphoreType.DMA],
    )(_sc_body)
    return f(p2d, t2d, params)


def _tri(n, cmp):
    a = lax.broadcasted_iota(jnp.int32, (n, n), 0)
    b = lax.broadcasted_iota(jnp.int32, (n, n), 1)
    return cmp(a, b).astype(jnp.float32)


def _mm(a, b):
    return jnp.dot(a, b, precision=lax.Precision.HIGHEST,
                   preferred_element_type=jnp.float32)


def _fin_body(hist_ref, par_ref, diff_ref, exc_ref):
    h = par_ref[0]
    vmin = par_ref[1]
    H = jnp.sum(hist_ref[...], axis=0)
    U = _tri(BL, lambda a, b: a <= b)
    SL = _tri(BR, lambda a, b: b < a)
    P = _tri(BL, lambda a, b: b == a + 1)
    SD = _tri(BR, lambda a, b: b == a - 1)
    col0 = (lax.broadcasted_iota(jnp.int32, (1, BL), 1) == 0).astype(jnp.float32)
    jj = (lax.broadcasted_iota(jnp.int32, (BR, BL), 0) * BL
          + lax.broadcasted_iota(jnp.int32, (BR, BL), 1)).astype(jnp.float32)
    x_up = vmin + (jj + 1.0) * h

    def cell_int(cnt, vsum):
        incl = _mm(cnt, U)
        rs = incl[:, BL - 1:BL]
        off = _mm(SL, rs)
        c_ex = incl - cnt + off
        return h * c_ex + x_up * cnt - vsum

    def shift1(a):
        carry = _mm(SD, a[:, BL - 1:BL])
        return _mm(a, P) + carry * col0

    ap = cell_int(H[0], H[1])
    at = cell_int(H[2], H[3])
    diff_ref[...] = jnp.sum(jnp.abs(ap - at), keepdims=True)
    exc_ref[...] = (jnp.sum(jnp.maximum(shift1(at) - ap, 0.0), keepdims=True) +
                    jnp.sum(jnp.maximum(shift1(ap) - at, 0.0), keepdims=True))


def _finalize(hist4d, par):
    out = jax.ShapeDtypeStruct((1, 1), jnp.float32)
    return pl.pallas_call(
        _fin_body,
        in_specs=[pl.BlockSpec(memory_space=pltpu.VMEM),
                  pl.BlockSpec(memory_space=pltpu.SMEM)],
        out_specs=[pl.BlockSpec(memory_space=pltpu.VMEM)] * 2,
        out_shape=[out] * 2,
    )(hist4d, par)


def kernel(output, target):
    p2d = output.reshape(ROWS, 1024)
    t2d = target.reshape(ROWS, 1024)
    scal, params = _stats(p2d, t2d)
    hist = _sc_hist(p2d, t2d, params.reshape(64))
    sd, se = _finalize(hist.reshape(NT, 4, BR, BL), scal[2:4])
    nrmse = scal[0]
    scaling = scal[1]
    return (scaling * (sd[0, 0] + se[0, 0]) + 2.0 * nrmse) / 4.0


# speedup vs baseline: 1.1405x; 1.1405x over previous
import functools
import math

import jax
import jax.numpy as jnp
from jax import lax
from jax.experimental import pallas as pl
from jax.experimental.pallas import tpu as pltpu
from jax.experimental.pallas import tpu_sc as plsc

N = 8388608
EPS = 1e-6
B = 1792
BR, BL = 14, 128
B16 = B * 16
NT = 32
ROWS = 8192
NROW = ROWS // NT
RCH = 16
NCH = NROW // RCH
BLK = 512
GRID1 = ROWS // BLK
CF = float(1.63 * math.sqrt(8388608.0))


def _stats_body(p_ref, t_ref, scal_ref, par_ref,
                a_t, a_sq, a_mx, a_mn):
    i = pl.program_id(0)
    p = p_ref[...]
    t = t_ref[...]
    d = p - t
    bs_t = jnp.sum(t, axis=0, keepdims=True)
    bs_sq = jnp.sum(d * d, axis=0, keepdims=True)
    b_mx = jnp.max(jnp.maximum(p, t), axis=0, keepdims=True)
    b_mn = jnp.min(jnp.minimum(p, t), axis=0, keepdims=True)

    @pl.when(i == 0)
    def _():
        a_t[...] = bs_t
        a_sq[...] = bs_sq
        a_mx[...] = b_mx
        a_mn[...] = b_mn

    @pl.when(i > 0)
    def _():
        a_t[...] = a_t[...] + bs_t
        a_sq[...] = a_sq[...] + bs_sq
        a_mx[...] = jnp.maximum(a_mx[...], b_mx)
        a_mn[...] = jnp.minimum(a_mn[...], b_mn)

    @pl.when(i == pl.num_programs(0) - 1)
    def _():
        sum_t = jnp.sum(a_t[...])
        mse = jnp.sum(a_sq[...]) / N
        vmax = jnp.max(a_mx[...])
        vmin = jnp.min(a_mn[...])
        nrmse = jnp.sqrt(mse) / (sum_t / N + EPS)
        vc = vmax - vmin + EPS
        c = jnp.float32(CF) * vc / N
        scal_ref[0] = nrmse
        scal_ref[1] = 100.0 / (jnp.float32(CF) * vc)
        scal_ref[2] = c
        scal_ref[3] = vmin
        ii = lax.broadcasted_iota(jnp.int32, (1, 64), 1)
        par_ref[...] = jnp.where(ii < 16, vmin,
                                 jnp.where(ii < 32, 1.0 / c, 0.0))


def _stats(p2d, t2d):
    return pl.pallas_call(
        _stats_body,
        grid=(GRID1,),
        in_specs=[pl.BlockSpec((BLK, 1024), lambda i: (i, 0)),
                  pl.BlockSpec((BLK, 1024), lambda i: (i, 0))],
        out_specs=[pl.BlockSpec((4,), lambda i: (0,), memory_space=pltpu.SMEM),
                   pl.BlockSpec((1, 64), lambda i: (0, 0))],
        out_shape=[jax.ShapeDtypeStruct((4,), jnp.float32),
                   jax.ShapeDtypeStruct((1, 64), jnp.float32)],
        scratch_shapes=[pltpu.VMEM((1, 1024), jnp.float32)] * 4,
    )(p2d, t2d)


def _sc_body(p_hbm, t_hbm, par_hbm, out_hbm,
             hp, ht, buf0, buf1, fold_v, par_v, sem0, sem1):
    wid = lax.axis_index("s") * 2 + lax.axis_index("c")
    row0 = wid * NROW
    pltpu.sync_copy(par_hbm, par_v)
    vmin = par_v[pl.ds(0, 16)]
    vinvh = par_v[pl.ds(16, 16)]
    lane = lax.iota(jnp.int32, 16)
    ones = jnp.full((16,), 1.0, jnp.float32)
    zeros = jnp.zeros((16,), jnp.float32)

    def zbody(j, c):
        for u in range(4):
            hp[pl.ds(j * 64 + u * 16, 16)] = zeros
            ht[pl.ds(j * 64 + u * 16, 16)] = zeros
        return c

    lax.fori_loop(0, B16 // 64, zbody, 0)

    def compute(buf, hist):
        def vbody(k, c):
            vs = [buf[r, pl.ds(k * 16, 16)] for r in range(RCH)]
            ads = []
            for v in vs:
                idx = lax.convert_element_type((v - vmin) * vinvh, jnp.int32)
                ads.append(idx * 16 + lane)
            for a in ads:
                plsc.addupdate_scatter(hist, [a], ones)
            return c

        lax.fori_loop(0, 1024 // 16, vbody, 0)

    def src(x_hbm, ci):
        return x_hbm.at[pl.ds(row0 + ci * RCH, RCH), :]

    def process(x_hbm, hist):
        pltpu.make_async_copy(src(x_hbm, 0), buf0, sem0).start()
        pltpu.make_async_copy(src(x_hbm, 1), buf1, sem1).start()

        def cbody(cj, c):
            c0 = cj * 2
            pltpu.make_async_copy(src(x_hbm, c0), buf0, sem0).wait()
            compute(buf0, hist)

            @pl.when(c0 + 2 < NCH)
            def _():
                pltpu.make_async_copy(src(x_hbm, c0 + 2), buf0, sem0).start()

            pltpu.make_async_copy(src(x_hbm, c0 + 1), buf1, sem1).wait()
            compute(buf1, hist)

            @pl.when(c0 + 3 < NCH)
            def _():
                pltpu.make_async_copy(src(x_hbm, c0 + 3), buf1, sem1).start()

            return c

        lax.fori_loop(0, NCH // 2, cbody, 0)

    process(p_hbm, hp)
    process(t_hbm, ht)

    def fbody(j0, c):
        bkt = (j0 * 16 + lane) * 16
        pc = [plsc.load_gather(hp, [bkt + l]) for l in range(16)]
        ps = [plsc.load_gather(ht, [bkt + l]) for l in range(16)]

        def tree(xs):
            while len(xs) > 1:
                xs = [a + b for a, b in zip(xs[::2], xs[1::2])]
            return xs[0]

        fold_v[0, pl.ds(j0 * 16, 16)] = tree(pc)
        fold_v[1, pl.ds(j0 * 16, 16)] = tree(ps)
        return c

    lax.fori_loop(0, B // 16, fbody, 0)
    pltpu.sync_copy(fold_v, out_hbm.at[wid])


def _sc_hist(p2d, t2d, params):
    mesh = plsc.VectorSubcoreMesh(core_axis_name="c", subcore_axis_name="s")
    f = functools.partial(
        pl.kernel,
        mesh=mesh,
        out_type=jax.ShapeDtypeStruct((NT, 2, B), jnp.float32),
        compiler_params=pltpu.CompilerParams(needs_layout_passes=False),
        scratch_types=[pltpu.VMEM((B16,), jnp.float32),
                       pltpu.VMEM((B16,), jnp.float32),
                       pltpu.VMEM((RCH, 1024), jnp.float32),
                       pltpu.VMEM((RCH, 1024), jnp.float32),
                       pltpu.VMEM((2, B), jnp.float32),
                       pltpu.VMEM((64,), jnp.float32),
                       pltpu.SemaphoreType.DMA,
                       pltpu.SemaphoreType.DMA],
    )(_sc_body)
    return f(p2d, t2d, params)


def _tri(n, cmp):
    a = lax.broadcasted_iota(jnp.int32, (n, n), 0)
    b = lax.broadcasted_iota(jnp.int32, (n, n), 1)
    return cmp(a, b).astype(jnp.float32)


def _mm(a, b):
    return jnp.dot(a, b, precision=lax.Precision.HIGHEST,
                   preferred_element_type=jnp.float32)


def _fin_body(hist_ref, par_ref, diff_ref, exc_ref):
    h = par_ref[0]
    H = jnp.sum(hist_ref[...], axis=0)
    U = _tri(BL, lambda a, b: a <= b)
    SL = _tri(BR, lambda a, b: b < a)
    P = _tri(BL, lambda a, b: b == a + 1)
    SD = _tri(BR, lambda a, b: b == a - 1)
    col0 = (lax.broadcasted_iota(jnp.int32, (1, BL), 1) == 0).astype(jnp.float32)

    def cell_int(cnt):
        incl = _mm(cnt, U)
        rs = incl[:, BL - 1:BL]
        off = _mm(SL, rs)
        return h * (incl - 0.5 * cnt + off)

    def shift1(a):
        carry = _mm(SD, a[:, BL - 1:BL])
        return _mm(a, P) + carry * col0

    ap = cell_int(H[0])
    at = cell_int(H[1])
    diff_ref[...] = jnp.sum(jnp.abs(ap - at), keepdims=True)
    exc_ref[...] = (jnp.sum(jnp.maximum(shift1(at) - ap, 0.0), keepdims=True) +
                    jnp.sum(jnp.maximum(shift1(ap) - at, 0.0), keepdims=True))


def _finalize(hist4d, par):
    out = jax.ShapeDtypeStruct((1, 1), jnp.float32)
    return pl.pallas_call(
        _fin_body,
        in_specs=[pl.BlockSpec(memory_space=pltpu.VMEM),
                  pl.BlockSpec(memory_space=pltpu.SMEM)],
        out_specs=[pl.BlockSpec(memory_space=pltpu.VMEM)] * 2,
        out_shape=[out] * 2,
    )(hist4d, par)


def kernel(output, target):
    p2d = output.reshape(ROWS, 1024)
    t2d = target.reshape(ROWS, 1024)
    scal, params = _stats(p2d, t2d)
    hist = _sc_hist(p2d, t2d, params.reshape(64))
    sd, se = _finalize(hist.reshape(NT, 2, BR, BL), scal[2:3])
    nrmse = scal[0]
    scaling = scal[1]
    return (scaling * (sd[0, 0] + se[0, 0]) + 2.0 * nrmse) / 4.0


# speedup vs baseline: 1.1681x; 1.0241x over previous
import functools
import math

import jax
import jax.numpy as jnp
from jax import lax
from jax.experimental import pallas as pl
from jax.experimental.pallas import tpu as pltpu
from jax.experimental.pallas import tpu_sc as plsc

N = 8388608
EPS = 1e-6
B = 1792
BR, BL = 14, 128
B16 = B * 16
NT = 32
ROWS = 8192
NROW = ROWS // NT
RCH = 8
NCH = NROW // RCH
BLK = 512
GRID1 = ROWS // BLK
CF = float(1.63 * math.sqrt(8388608.0))


def _stats_body(p_ref, t_ref, scal_ref, par_ref,
                a_t, a_sq, a_mx, a_mn):
    i = pl.program_id(0)
    p = p_ref[...]
    t = t_ref[...]
    d = p - t
    bs_t = jnp.sum(t, axis=0, keepdims=True)
    bs_sq = jnp.sum(d * d, axis=0, keepdims=True)
    b_mx = jnp.max(jnp.maximum(p, t), axis=0, keepdims=True)
    b_mn = jnp.min(jnp.minimum(p, t), axis=0, keepdims=True)

    @pl.when(i == 0)
    def _():
        a_t[...] = bs_t
        a_sq[...] = bs_sq
        a_mx[...] = b_mx
        a_mn[...] = b_mn

    @pl.when(i > 0)
    def _():
        a_t[...] = a_t[...] + bs_t
        a_sq[...] = a_sq[...] + bs_sq
        a_mx[...] = jnp.maximum(a_mx[...], b_mx)
        a_mn[...] = jnp.minimum(a_mn[...], b_mn)

    @pl.when(i == pl.num_programs(0) - 1)
    def _():
        sum_t = jnp.sum(a_t[...])
        mse = jnp.sum(a_sq[...]) / N
        vmax = jnp.max(a_mx[...])
        vmin = jnp.min(a_mn[...])
        nrmse = jnp.sqrt(mse) / (sum_t / N + EPS)
        vc = vmax - vmin + EPS
        c = jnp.float32(CF) * vc / N
        scal_ref[0] = nrmse
        scal_ref[1] = 100.0 / (jnp.float32(CF) * vc)
        scal_ref[2] = c
        scal_ref[3] = vmin
        ii = lax.broadcasted_iota(jnp.int32, (1, 64), 1)
        par_ref[...] = jnp.where(ii < 16, vmin,
                                 jnp.where(ii < 32, 1.0 / c, 0.0))


def _stats(p2d, t2d):
    return pl.pallas_call(
        _stats_body,
        grid=(GRID1,),
        in_specs=[pl.BlockSpec((BLK, 1024), lambda i: (i, 0)),
                  pl.BlockSpec((BLK, 1024), lambda i: (i, 0))],
        out_specs=[pl.BlockSpec((4,), lambda i: (0,), memory_space=pltpu.SMEM),
                   pl.BlockSpec((1, 64), lambda i: (0, 0))],
        out_shape=[jax.ShapeDtypeStruct((4,), jnp.float32),
                   jax.ShapeDtypeStruct((1, 64), jnp.float32)],
        scratch_shapes=[pltpu.VMEM((1, 1024), jnp.float32)] * 4,
    )(p2d, t2d)


def _sc_body(p_hbm, t_hbm, par_hbm, out_hbm,
             hp, ht, bp0, bp1, bt0, bt1, fold_v, par_v,
             semp0, semp1, semt0, semt1):
    wid = lax.axis_index("s") * 2 + lax.axis_index("c")
    row0 = wid * NROW
    pltpu.sync_copy(par_hbm, par_v)
    vmin = par_v[pl.ds(0, 16)]
    vinvh = par_v[pl.ds(16, 16)]
    lane = lax.iota(jnp.int32, 16)
    ones = jnp.full((16,), 1.0, jnp.float32)
    zeros = jnp.zeros((16,), jnp.float32)

    def zbody(j, c):
        for u in range(4):
            hp[pl.ds(j * 64 + u * 16, 16)] = zeros
            ht[pl.ds(j * 64 + u * 16, 16)] = zeros
        return c

    lax.fori_loop(0, B16 // 64, zbody, 0)

    def compute(bufp, buft):
        def vbody(k, c):
            vps = [bufp[r, pl.ds(k * 16, 16)] for r in range(RCH)]
            vts = [buft[r, pl.ds(k * 16, 16)] for r in range(RCH)]

            def addr(v):
                idx = lax.convert_element_type((v - vmin) * vinvh, jnp.int32)
                return idx * 16 + lane

            aps = [addr(v) for v in vps]
            ats = [addr(v) for v in vts]
            for ap, at in zip(aps, ats):
                plsc.addupdate_scatter(hp, [ap], ones)
                plsc.addupdate_scatter(ht, [at], ones)
            return c

        lax.fori_loop(0, 1024 // 16, vbody, 0)

    def srcp(ci):
        return p_hbm.at[pl.ds(row0 + ci * RCH, RCH), :]

    def srct(ci):
        return t_hbm.at[pl.ds(row0 + ci * RCH, RCH), :]

    pltpu.make_async_copy(srcp(0), bp0, semp0).start()
    pltpu.make_async_copy(srct(0), bt0, semt0).start()
    pltpu.make_async_copy(srcp(1), bp1, semp1).start()
    pltpu.make_async_copy(srct(1), bt1, semt1).start()

    def cbody(cj, c):
        c0 = cj * 2
        pltpu.make_async_copy(srcp(c0), bp0, semp0).wait()
        pltpu.make_async_copy(srct(c0), bt0, semt0).wait()
        compute(bp0, bt0)

        @pl.when(c0 + 2 < NCH)
        def _():
            pltpu.make_async_copy(srcp(c0 + 2), bp0, semp0).start()
            pltpu.make_async_copy(srct(c0 + 2), bt0, semt0).start()

        pltpu.make_async_copy(srcp(c0 + 1), bp1, semp1).wait()
        pltpu.make_async_copy(srct(c0 + 1), bt1, semt1).wait()
        compute(bp1, bt1)

        @pl.when(c0 + 3 < NCH)
        def _():
            pltpu.make_async_copy(srcp(c0 + 3), bp1, semp1).start()
            pltpu.make_async_copy(srct(c0 + 3), bt1, semt1).start()

        return c

    lax.fori_loop(0, NCH // 2, cbody, 0)

    def fbody(j0, c):
        bkt = (j0 * 16 + lane) * 16
        pc = [plsc.load_gather(hp, [bkt + l]) for l in range(16)]
        ps = [plsc.load_gather(ht, [bkt + l]) for l in range(16)]

        def tree(xs):
            while len(xs) > 1:
                xs = [a + b for a, b in zip(xs[::2], xs[1::2])]
            return xs[0]

        fold_v[0, pl.ds(j0 * 16, 16)] = tree(pc)
        fold_v[1, pl.ds(j0 * 16, 16)] = tree(ps)
        return c

    lax.fori_loop(0, B // 16, fbody, 0)
    pltpu.sync_copy(fold_v, out_hbm.at[wid])


def _sc_hist(p2d, t2d, params):
    mesh = plsc.VectorSubcoreMesh(core_axis_name="c", subcore_axis_name="s")
    f = functools.partial(
        pl.kernel,
        mesh=mesh,
        out_type=jax.ShapeDtypeStruct((NT, 2, B), jnp.float32),
        compiler_params=pltpu.CompilerParams(needs_layout_passes=False),
        scratch_types=[pltpu.VMEM((B16,), jnp.float32),
                       pltpu.VMEM((B16,), jnp.float32),
                       pltpu.VMEM((RCH, 1024), jnp.float32),
                       pltpu.VMEM((RCH, 1024), jnp.float32),
                       pltpu.VMEM((RCH, 1024), jnp.float32),
                       pltpu.VMEM((RCH, 1024), jnp.float32),
                       pltpu.VMEM((2, B), jnp.float32),
                       pltpu.VMEM((64,), jnp.float32),
                       pltpu.SemaphoreType.DMA,
                       pltpu.SemaphoreType.DMA,
                       pltpu.SemaphoreType.DMA,
                       pltpu.SemaphoreType.DMA],
    )(_sc_body)
    return f(p2d, t2d, params)


def _tri(n, cmp):
    a = lax.broadcasted_iota(jnp.int32, (n, n), 0)
    b = lax.broadcasted_iota(jnp.int32, (n, n), 1)
    return cmp(a, b).astype(jnp.float32)


def _mm(a, b):
    return jnp.dot(a, b, precision=lax.Precision.HIGHEST,
                   preferred_element_type=jnp.float32)


def _fin_body(hist_ref, par_ref, diff_ref, exc_ref):
    h = par_ref[0]
    H = jnp.sum(hist_ref[...], axis=0)
    U = _tri(BL, lambda a, b: a <= b)
    SL = _tri(BR, lambda a, b: b < a)
    P = _tri(BL, lambda a, b: b == a + 1)
    SD = _tri(BR, lambda a, b: b == a - 1)
    col0 = (lax.broadcasted_iota(jnp.int32, (1, BL), 1) == 0).astype(jnp.float32)

    def cell_int(cnt):
        incl = _mm(cnt, U)
        rs = incl[:, BL - 1:BL]
        off = _mm(SL, rs)
        return h * (incl - 0.5 * cnt + off)

    def shift1(a):
        carry = _mm(SD, a[:, BL - 1:BL])
        return _mm(a, P) + carry * col0

    ap = cell_int(H[0])
    at = cell_int(H[1])
    diff_ref[...] = jnp.sum(jnp.abs(ap - at), keepdims=True)
    exc_ref[...] = (jnp.sum(jnp.maximum(shift1(at) - ap, 0.0), keepdims=True) +
                    jnp.sum(jnp.maximum(shift1(ap) - at, 0.0), keepdims=True))


def _finalize(hist4d, par):
    out = jax.ShapeDtypeStruct((1, 1), jnp.float32)
    return pl.pallas_call(
        _fin_body,
        in_specs=[pl.BlockSpec(memory_space=pltpu.VMEM),
                  pl.BlockSpec(memory_space=pltpu.SMEM)],
        out_specs=[pl.BlockSpec(memory_space=pltpu.VMEM)] * 2,
        out_shape=[out] * 2,
    )(hist4d, par)


def kernel(output, target):
    p2d = output.reshape(ROWS, 1024)
    t2d = target.reshape(ROWS, 1024)
    scal, params = _stats(p2d, t2d)
    hist = _sc_hist(p2d, t2d, params.reshape(64))
    sd, se = _finalize(hist.reshape(NT, 2, BR, BL), scal[2:3])
    nrmse = scal[0]
    scaling = scal[1]
    return (scaling * (sd[0, 0] + se[0, 0]) + 2.0 * nrmse) / 4.0
